# Initial kernel scaffold; baseline (speedup 1.0000x reference)
import jax
import jax.numpy as jnp
from jax.experimental import pallas as pl


def kernel(x, edge_index, batch, embed_table, W1, b1, W2, b2, Wo, bo):
    raise NotImplementedError("write your pallas kernel here")


# speedup vs baseline: 5.5796x; 5.5796x over previous
import functools

import jax
import jax.numpy as jnp
from jax import lax
from jax.experimental import pallas as pl
from jax.experimental.pallas import tpu as pltpu
from jax.experimental.pallas import tpu_sc as plsc

N = 10000
NP = 10240
E = 160000
EP = 163840
VOCAB = 100000
D = 300
H1 = 128
H2 = 64
H2P = 128
OUT = 6
G = 64
NC, NS = 2, 16
NW = NC * NS
DUMMY = N

ROW_BLK = 1024
GRID = NP // ROW_BLK
VBLK = 2000
VGRID = VOCAB // VBLK

_mesh = functools.partial(
    plsc.VectorSubcoreMesh, core_axis_name="c", subcore_axis_name="s")


def _wid():
    return lax.axis_index("s") * NC + lax.axis_index("c")


def _tc0_body(t, w1, tw):
    tw[...] = jnp.dot(t[...], w1[...], preferred_element_type=jnp.float32)


_tc0 = pl.pallas_call(
    _tc0_body,
    grid=(VGRID,),
    in_specs=[
        pl.BlockSpec((VBLK, D), lambda i: (i, 0)),
        pl.BlockSpec((D, H1), lambda i: (0, 0)),
    ],
    out_specs=pl.BlockSpec((VBLK, H1), lambda i: (i, 0)),
    out_shape=jax.ShapeDtypeStruct((VOCAB, H1), jnp.float32),
)


def _embed_body(tw, idx, out, idx_v, rows_v, sem):
    w = _wid()

    def chunk(j, carry):
        base = w * (NP // NW) + j * 80
        pltpu.sync_copy(idx.at[pl.ds(base, 80)], idx_v)
        pltpu.async_copy(tw.at[idx_v], rows_v, sem).wait()
        pltpu.sync_copy(rows_v, out.at[pl.ds(base, 80)])
        return carry

    lax.fori_loop(0, NP // NW // 80, chunk, 0)


_embed = pl.kernel(
    _embed_body,
    out_type=jax.ShapeDtypeStruct((NP, H1), jnp.float32),
    mesh=_mesh(),
    scratch_types=[
        pltpu.VMEM((80,), jnp.int32),
        pltpu.VMEM((80, H1), jnp.float32),
        pltpu.SemaphoreType.DMA,
    ],
)


def _deg_body(dst, out, ones_v, zeros_v, didx_v, shared):
    cid = lax.axis_index("c")
    sid = lax.axis_index("s")
    rps = NP // NS

    ones16 = jnp.ones((16,), jnp.float32)
    zeros16 = jnp.zeros((16,), jnp.float32)

    def ofill(i, c):
        ones_v[pl.ds(i * 16, 16)] = ones16
        return c

    lax.fori_loop(0, 128 // 16, ofill, 0)

    def zfill(i, c):
        zeros_v[pl.ds(i * 16, 16)] = zeros16
        return c

    lax.fori_loop(0, rps // 16, zfill, 0)

    pltpu.sync_copy(zeros_v, shared.at[pl.ds(sid * rps, rps)])
    plsc.subcore_barrier()

    eps_core = EP // NC
    eps_sub = eps_core // NS

    def chunk(j, c):
        base = cid * eps_core + sid * eps_sub + j * 128
        pltpu.sync_copy(dst.at[pl.ds(base, 128)], didx_v)
        pltpu.sync_copy(ones_v, shared.at[didx_v], add=True)
        return c

    lax.fori_loop(0, eps_sub // 128, chunk, 0)
    plsc.subcore_barrier()
    pltpu.sync_copy(shared.at[pl.ds(sid * rps, rps)],
                    out.at[pl.ds(cid * NP + sid * rps, rps)])


_deg = pl.kernel(
    _deg_body,
    out_type=jax.ShapeDtypeStruct((NC * NP,), jnp.float32),
    mesh=_mesh(),
    scratch_types=[
        pltpu.VMEM((128,), jnp.float32),
        pltpu.VMEM((NP // NS,), jnp.float32),
        pltpu.VMEM((128,), jnp.int32),
        pltpu.VMEM_SHARED((NP,), jnp.float32),
    ],
)


def _agg_body(g, src, dst, out, shared, sidx_v, didx_v, rows_v, sem):
    cid = lax.axis_index("c")
    sid = lax.axis_index("s")
    rps = NP // NS

    pltpu.sync_copy(g.at[pl.ds(sid * rps, rps)], shared.at[pl.ds(sid * rps, rps)])
    plsc.subcore_barrier()

    eps_core = EP // NC
    eps_sub = eps_core // NS

    def chunk(j, carry):
        base = cid * eps_core + sid * eps_sub + j * 128
        pltpu.sync_copy(src.at[pl.ds(base, 128)], sidx_v)
        pltpu.async_copy(g.at[sidx_v], rows_v, sem).wait()
        pltpu.sync_copy(dst.at[pl.ds(base, 128)], didx_v)
        pltpu.sync_copy(rows_v, shared.at[didx_v], add=True)
        return carry

    lax.fori_loop(0, eps_sub // 128, chunk, 0)
    plsc.subcore_barrier()
    pltpu.sync_copy(shared.at[pl.ds(sid * rps, rps)],
                    out.at[pl.ds(cid * NP + sid * rps, rps)])


_agg = pl.kernel(
    _agg_body,
    out_type=jax.ShapeDtypeStruct((NC * NP, H1), jnp.float32),
    mesh=_mesh(),
    scratch_types=[
        pltpu.VMEM_SHARED((NP, H1), jnp.float32),
        pltpu.VMEM((128,), jnp.int32),
        pltpu.VMEM((128,), jnp.int32),
        pltpu.VMEM((128, H1), jnp.float32),
        pltpu.SemaphoreType.DMA,
    ],
)


def _tc1_body(z, degp, g1, dinv):
    deg = jnp.sum(degp[...], axis=1, keepdims=True) + 1.0
    di = lax.rsqrt(deg)
    g1[...] = z[...] * di
    dinv[...] = di


_tc1 = pl.pallas_call(
    _tc1_body,
    grid=(GRID,),
    in_specs=[
        pl.BlockSpec((ROW_BLK, H1), lambda i: (i, 0)),
        pl.BlockSpec((ROW_BLK, NC), lambda i: (i, 0)),
    ],
    out_specs=[
        pl.BlockSpec((ROW_BLK, H1), lambda i: (i, 0)),
        pl.BlockSpec((ROW_BLK, 1), lambda i: (i, 0)),
    ],
    out_shape=[
        jax.ShapeDtypeStruct((NP, H1), jnp.float32),
        jax.ShapeDtypeStruct((NP, 1), jnp.float32),
    ],
)


def _tc2_body(a0, a1, g1, dinv, b1, w2, g2):
    di = dinv[...]
    agg = a0[...] + a1[...] - g1[...]
    h1 = jnp.maximum(agg * di + b1[...], 0.0)
    g2[...] = jnp.dot(h1, w2[...], preferred_element_type=jnp.float32) * di


_tc2 = pl.pallas_call(
    _tc2_body,
    grid=(GRID,),
    in_specs=[
        pl.BlockSpec((ROW_BLK, H1), lambda i: (i, 0)),
        pl.BlockSpec((ROW_BLK, H1), lambda i: (i, 0)),
        pl.BlockSpec((ROW_BLK, H1), lambda i: (i, 0)),
        pl.BlockSpec((ROW_BLK, 1), lambda i: (i, 0)),
        pl.BlockSpec((1, H1), lambda i: (0, 0)),
        pl.BlockSpec((H1, H2P), lambda i: (0, 0)),
    ],
    out_specs=pl.BlockSpec((ROW_BLK, H2P), lambda i: (i, 0)),
    out_shape=jax.ShapeDtypeStruct((NP, H2P), jnp.float32),
)


def _tc3_body(a0, a1, g2, dinv, b2, bat, wo, bo, out):
    di = dinv[...]
    agg = a0[...] + a1[...] - g2[...]
    h2 = jnp.maximum(agg * di + b2[...], 0.0)
    onehot = (bat[...] == lax.broadcasted_iota(jnp.int32, (NP, G), 1))
    onehot = onehot.astype(jnp.float32)
    sums = lax.dot_general(onehot, h2, (((0,), (0,)), ((), ())),
                           preferred_element_type=jnp.float32)
    counts = jnp.sum(onehot, axis=0)
    pooled = sums / jnp.maximum(counts, 1.0)[:, None]
    o = jnp.dot(pooled, wo[...], preferred_element_type=jnp.float32) + bo[...]
    out[...] = jnp.maximum(o, 0.0)


_tc3 = pl.pallas_call(
    _tc3_body,
    out_shape=jax.ShapeDtypeStruct((G, OUT), jnp.float32),
)


def kernel(x, edge_index, batch, embed_table, W1, b1, W2, b2, Wo, bo):
    idx = jnp.concatenate(
        [x[:, 0].astype(jnp.int32), jnp.zeros((NP - N,), jnp.int32)])
    src = jnp.concatenate(
        [edge_index[0].astype(jnp.int32), jnp.zeros((EP - E,), jnp.int32)])
    dst = jnp.concatenate(
        [edge_index[1].astype(jnp.int32), jnp.full((EP - E,), DUMMY, jnp.int32)])
    bat = jnp.concatenate(
        [batch.astype(jnp.int32), jnp.full((NP - N,), G, jnp.int32)])[:, None]
    W2p = jnp.pad(W2, ((0, 0), (0, H2P - H2)))
    b2p = jnp.pad(b2, (0, H2P - H2)).reshape(1, H2P)
    Wop = jnp.pad(Wo, ((0, H2P - H2), (0, 0)))

    TW = _tc0(embed_table, W1)
    z = _embed(TW, idx)
    degp = _deg(dst).reshape(NC, NP).T
    g1, dinv = _tc1(z, degp)
    aggf = _agg(g1, src, dst)
    g2 = _tc2(aggf[:NP], aggf[NP:], g1, dinv,
              b1.reshape(1, H1), W2p)
    aggf2 = _agg(g2, src, dst)
    out = _tc3(aggf2[:NP], aggf2[NP:], g2, dinv,
               b2p, bat, Wop, bo.reshape(1, OUT))
    return out


# speedup vs baseline: 7.0135x; 1.2570x over previous
import functools

import jax
import jax.numpy as jnp
from jax import lax
from jax.experimental import pallas as pl
from jax.experimental.pallas import tpu as pltpu
from jax.experimental.pallas import tpu_sc as plsc

N = 10000
NP = 10240
E = 160000
EP = 163840
VOCAB = 100000
D = 300
H1 = 128
H2 = 64
H2P = 128
OUT = 6
G = 64
NC, NS = 2, 16
NW = NC * NS
DUMMY = N

ROW_BLK = 1024
GRID = NP // ROW_BLK
VBLK = 2000
VGRID = VOCAB // VBLK

_mesh = functools.partial(
    plsc.VectorSubcoreMesh, core_axis_name="c", subcore_axis_name="s")


def _wid():
    return lax.axis_index("s") * NC + lax.axis_index("c")


def _tc0_body(t, w1, tw):
    tw[...] = jnp.dot(t[...], w1[...], preferred_element_type=jnp.float32)


_tc0 = pl.pallas_call(
    _tc0_body,
    grid=(VGRID,),
    in_specs=[
        pl.BlockSpec((VBLK, D), lambda i: (i, 0)),
        pl.BlockSpec((D, H1), lambda i: (0, 0)),
    ],
    out_specs=pl.BlockSpec((VBLK, H1), lambda i: (i, 0)),
    out_shape=jax.ShapeDtypeStruct((VOCAB, H1), jnp.float32),
)


def _embed_body(tw, idx, out, idx_v, rows_v, sem):
    w = _wid()

    def chunk(j, carry):
        base = w * (NP // NW) + j * 80
        pltpu.sync_copy(idx.at[pl.ds(base, 80)], idx_v)
        pltpu.async_copy(tw.at[idx_v], rows_v, sem).wait()
        pltpu.sync_copy(rows_v, out.at[pl.ds(base, 80)])
        return carry

    lax.fori_loop(0, NP // NW // 80, chunk, 0)


_embed = pl.kernel(
    _embed_body,
    out_type=jax.ShapeDtypeStruct((NP, H1), jnp.float32),
    mesh=_mesh(),
    scratch_types=[
        pltpu.VMEM((80,), jnp.int32),
        pltpu.VMEM((80, H1), jnp.float32),
        pltpu.SemaphoreType.DMA,
    ],
)


def _deg_body(dst, out, ones_v, zeros_v, didx_v, shared):
    cid = lax.axis_index("c")
    sid = lax.axis_index("s")
    rps = NP // NS

    ones16 = jnp.ones((16,), jnp.float32)
    zeros16 = jnp.zeros((16,), jnp.float32)

    def ofill(i, c):
        ones_v[pl.ds(i * 16, 16)] = ones16
        return c

    lax.fori_loop(0, 128 // 16, ofill, 0)

    def zfill(i, c):
        zeros_v[pl.ds(i * 16, 16)] = zeros16
        return c

    lax.fori_loop(0, rps // 16, zfill, 0)

    pltpu.sync_copy(zeros_v, shared.at[pl.ds(sid * rps, rps)])
    plsc.subcore_barrier()

    eps_core = EP // NC
    eps_sub = eps_core // NS

    def chunk(j, c):
        base = cid * eps_core + sid * eps_sub + j * 128
        pltpu.sync_copy(dst.at[pl.ds(base, 128)], didx_v)
        pltpu.sync_copy(ones_v, shared.at[didx_v], add=True)
        return c

    lax.fori_loop(0, eps_sub // 128, chunk, 0)
    plsc.subcore_barrier()
    pltpu.sync_copy(shared.at[pl.ds(sid * rps, rps)],
                    out.at[pl.ds(cid * NP + sid * rps, rps)])


_deg = pl.kernel(
    _deg_body,
    out_type=jax.ShapeDtypeStruct((NC * NP,), jnp.float32),
    mesh=_mesh(),
    scratch_types=[
        pltpu.VMEM((128,), jnp.float32),
        pltpu.VMEM((NP // NS,), jnp.float32),
        pltpu.VMEM((128,), jnp.int32),
        pltpu.VMEM_SHARED((NP,), jnp.float32),
    ],
)


NCH = EP // NC // NS // 128


def _agg_body(g, src2, dst2, out, shared, sidx_v, didx_v,
              rows0, rows1, sem0, sem1):
    cid = lax.axis_index("c")
    sid = lax.axis_index("s")
    rps = NP // NS

    pltpu.sync_copy(g.at[pl.ds(sid * rps, rps)], shared.at[pl.ds(sid * rps, rps)])

    cbase = cid * (EP // NC // 128) + sid * NCH
    pltpu.sync_copy(src2.at[pl.ds(cbase, NCH)], sidx_v)
    pltpu.sync_copy(dst2.at[pl.ds(cbase, NCH)], didx_v)
    plsc.subcore_barrier()

    rows = (rows0, rows1)
    sems = (sem0, sem1)
    pltpu.async_copy(g.at[sidx_v.at[0]], rows0, sem0)
    pltpu.async_copy(g.at[sidx_v.at[1]], rows1, sem1)

    def pair(p, carry):
        for b in range(2):
            j = p * 2 + b
            pltpu.make_async_copy(g.at[sidx_v.at[j]], rows[b], sems[b]).wait()
            pltpu.sync_copy(rows[b], shared.at[didx_v.at[j]], add=True)

            @pl.when(j + 2 < NCH)
            def _():
                pltpu.async_copy(g.at[sidx_v.at[j + 2]], rows[b], sems[b])
        return carry

    lax.fori_loop(0, NCH // 2, pair, 0)
    plsc.subcore_barrier()
    pltpu.sync_copy(shared.at[pl.ds(sid * rps, rps)],
                    out.at[pl.ds(cid * NP + sid * rps, rps)])


_agg = pl.kernel(
    _agg_body,
    out_type=jax.ShapeDtypeStruct((NC * NP, H1), jnp.float32),
    mesh=_mesh(),
    scratch_types=[
        pltpu.VMEM_SHARED((NP, H1), jnp.float32),
        pltpu.VMEM((NCH, 128), jnp.int32),
        pltpu.VMEM((NCH, 128), jnp.int32),
        pltpu.VMEM((128, H1), jnp.float32),
        pltpu.VMEM((128, H1), jnp.float32),
        pltpu.SemaphoreType.DMA,
        pltpu.SemaphoreType.DMA,
    ],
)


def _tc1_body(z, degp, g1, dinv):
    deg = jnp.sum(degp[...], axis=1, keepdims=True) + 1.0
    di = lax.rsqrt(deg)
    g1[...] = z[...] * di
    dinv[...] = di


_tc1 = pl.pallas_call(
    _tc1_body,
    grid=(GRID,),
    in_specs=[
        pl.BlockSpec((ROW_BLK, H1), lambda i: (i, 0)),
        pl.BlockSpec((ROW_BLK, NC), lambda i: (i, 0)),
    ],
    out_specs=[
        pl.BlockSpec((ROW_BLK, H1), lambda i: (i, 0)),
        pl.BlockSpec((ROW_BLK, 1), lambda i: (i, 0)),
    ],
    out_shape=[
        jax.ShapeDtypeStruct((NP, H1), jnp.float32),
        jax.ShapeDtypeStruct((NP, 1), jnp.float32),
    ],
)


def _tc2_body(a0, a1, g1, dinv, b1, w2, g2):
    di = dinv[...]
    agg = a0[...] + a1[...] - g1[...]
    h1 = jnp.maximum(agg * di + b1[...], 0.0)
    g2[...] = jnp.dot(h1, w2[...], preferred_element_type=jnp.float32) * di


_tc2 = pl.pallas_call(
    _tc2_body,
    grid=(GRID,),
    in_specs=[
        pl.BlockSpec((ROW_BLK, H1), lambda i: (i, 0)),
        pl.BlockSpec((ROW_BLK, H1), lambda i: (i + GRID, 0)),
        pl.BlockSpec((ROW_BLK, H1), lambda i: (i, 0)),
        pl.BlockSpec((ROW_BLK, 1), lambda i: (i, 0)),
        pl.BlockSpec((1, H1), lambda i: (0, 0)),
        pl.BlockSpec((H1, H2P), lambda i: (0, 0)),
    ],
    out_specs=pl.BlockSpec((ROW_BLK, H2P), lambda i: (i, 0)),
    out_shape=jax.ShapeDtypeStruct((NP, H2P), jnp.float32),
)


def _tc3_body(af, g2, dinv, b2, bat, wo, bo, out):
    di = dinv[...]
    a = af[...]
    agg = a[:NP] + a[NP:] - g2[...]
    h2 = jnp.maximum(agg * di + b2[...], 0.0)
    onehot = (bat[...] == lax.broadcasted_iota(jnp.int32, (NP, G), 1))
    onehot = onehot.astype(jnp.float32)
    sums = lax.dot_general(onehot, h2, (((0,), (0,)), ((), ())),
                           preferred_element_type=jnp.float32)
    counts = jnp.sum(onehot, axis=0)
    pooled = sums / jnp.maximum(counts, 1.0)[:, None]
    o = jnp.dot(pooled, wo[...], preferred_element_type=jnp.float32) + bo[...]
    out[...] = jnp.maximum(o, 0.0)


_tc3 = pl.pallas_call(
    _tc3_body,
    out_shape=jax.ShapeDtypeStruct((G, OUT), jnp.float32),
)


def kernel(x, edge_index, batch, embed_table, W1, b1, W2, b2, Wo, bo):
    idx = jnp.concatenate(
        [x[:, 0].astype(jnp.int32), jnp.zeros((NP - N,), jnp.int32)])
    src = jnp.concatenate(
        [edge_index[0].astype(jnp.int32), jnp.zeros((EP - E,), jnp.int32)])
    dst = jnp.concatenate(
        [edge_index[1].astype(jnp.int32), jnp.full((EP - E,), DUMMY, jnp.int32)])
    bat = jnp.concatenate(
        [batch.astype(jnp.int32), jnp.full((NP - N,), G, jnp.int32)])[:, None]
    W2p = jnp.pad(W2, ((0, 0), (0, H2P - H2)))
    b2p = jnp.pad(b2, (0, H2P - H2)).reshape(1, H2P)
    Wop = jnp.pad(Wo, ((0, H2P - H2), (0, 0)))

    src2 = src.reshape(EP // 128, 128)
    dst2 = dst.reshape(EP // 128, 128)

    TW = _tc0(embed_table, W1)
    z = _embed(TW, idx)
    degp = _deg(dst).reshape(NC, NP).T
    g1, dinv = _tc1(z, degp)
    aggf = _agg(g1, src2, dst2)
    g2 = _tc2(aggf, aggf, g1, dinv,
              b1.reshape(1, H1), W2p)
    aggf2 = _agg(g2, src2, dst2)
    out = _tc3(aggf2, g2, dinv,
               b2p, bat, Wop, bo.reshape(1, OUT))
    return out
